# Initial kernel scaffold; baseline (speedup 1.0000x reference)
import jax
import jax.numpy as jnp
from jax.experimental import pallas as pl


def kernel(x, s_emb, t_emb, W1, b1, W2, b2, W3, b3):
    raise NotImplementedError("write your pallas kernel here")


# speedup vs baseline: 83.9915x; 83.9915x over previous
import jax
import jax.numpy as jnp
from jax.experimental import pallas as pl

B = 128
N = 512
L = 10
TOPK = 20
BL = B * L


def _body(xT_ref, s_ref, t_ref, W1_ref, W2_ref, W3_ref, b_ref, agg_ref):
    X = xT_ref[...]
    lane = jax.lax.broadcasted_iota(jnp.int32, (1, BL), 1) % L

    def shift(V, s):
        sh = jnp.concatenate([jnp.zeros((N, s), V.dtype), V[:, : BL - s]], axis=1)
        return jnp.where(lane >= s, sh, 0.0)

    def layer(V, W_ref, bias, d):
        acc = jnp.dot(W_ref[2], V, preferred_element_type=jnp.float32)
        acc += jnp.dot(W_ref[1], shift(V, d), preferred_element_type=jnp.float32)
        acc += jnp.dot(W_ref[0], shift(V, 2 * d), preferred_element_type=jnp.float32)
        return jax.nn.relu(acc + bias)

    b = b_ref[...]
    Z = layer(X, W1_ref, b[0][:, None], 1)
    Z = layer(Z, W2_ref, b[1][:, None], 2)
    Z = layer(Z, W3_ref, b[2][:, None], 4)
    Z = jax.nn.relu(Z + X)

    s = s_ref[...]
    t = t_ref[...]
    ns = s * jax.lax.rsqrt(jnp.sum(s * s, axis=1, keepdims=True))
    nt = t * jax.lax.rsqrt(jnp.sum(t * t, axis=1, keepdims=True))
    c = jax.nn.relu(jnp.dot(nt, ns.T, preferred_element_type=jnp.float32))
    col = jax.lax.broadcasted_iota(jnp.int32, (N, N), 1)
    row = jax.lax.broadcasted_iota(jnp.int32, (N, N), 0)
    S = jnp.where(col == row, -jnp.inf, c)

    A = jnp.zeros((N, N), jnp.float32)
    for _ in range(TOPK):
        v = jnp.max(S, axis=1, keepdims=True)
        m = S == v
        idx = jnp.where(m, col, N)
        jmin = jnp.min(idx, axis=1, keepdims=True)
        first = col == jmin
        A = jnp.where(first, 1.0, A)
        S = jnp.where(first, -jnp.inf, S)

    agg_ref[...] = jax.nn.relu(jnp.dot(A, Z, preferred_element_type=jnp.float32))


def kernel(x, s_emb, t_emb, W1, b1, W2, b2, W3, b3):
    xT = x.transpose(1, 0, 2).reshape(N, BL)
    Ws = [jnp.transpose(W, (2, 0, 1)) for W in (W1, W2, W3)]
    bs = jnp.stack([b1, b2, b3], axis=0)

    agg = pl.pallas_call(
        _body,
        out_shape=jax.ShapeDtypeStruct((N, BL), jnp.float32),
    )(xT, s_emb, t_emb, Ws[0], Ws[1], Ws[2], bs)

    out = agg.reshape(N, B, L).transpose(1, 0, 2).reshape(B * N, L)
    batch_s = jnp.tile(s_emb, (B, 1))
    batch_t = jnp.tile(t_emb, (B, 1))
    return jnp.concatenate([out, batch_s, batch_t], axis=-1)
